# baseline (speedup 1.0000x reference)
import functools

import jax
import jax.numpy as jnp
from jax import lax
from jax.experimental import pallas as pl
from jax.experimental.pallas import tpu as pltpu
from jax.experimental.pallas import tpu_sc as plsc

_EPS_BN = 1e-5
_EPS_DEN = 1e-6
_BN_SCALE = (1.0 / (1.0 + _EPS_BN)) ** 0.5

N = 10000
E = 160000
H = 128
NT = 32
EPT = E // NT
K = 40
NCH = EPT // K
DUMP0 = N
CARRY0 = ((N + 32 + 127) // 128) * 128
NROWS = CARRY0 + 128
NROWS2 = NROWS + NT * K
HV = H // 16
ZB = 64


def _mm_kernel(x_ref, w_ref, b_ref, o_ref):
    o_ref[...] = (
        jnp.dot(x_ref[...], w_ref[...], preferred_element_type=jnp.float32)
        + b_ref[...]
    )


def _mm(x, w, b, bn):
    n, k = x.shape
    m = w.shape[1]
    return pl.pallas_call(
        _mm_kernel,
        grid=(n // bn,),
        in_specs=[
            pl.BlockSpec((bn, k), lambda i: (i, 0)),
            pl.BlockSpec((k, m), lambda i: (0, 0)),
            pl.BlockSpec((1, m), lambda i: (0, 0)),
        ],
        out_specs=pl.BlockSpec((bn, m), lambda i: (i, 0)),
        out_shape=jax.ShapeDtypeStruct((n, m), jnp.float32),
    )(x, w, b.reshape(1, m))


def _node_update_kernel(ax_ref, num_ref, den_ref, cn_ref, cd_ref, bounds_ref,
                        xin_ref, g_ref, b_ref, o_ref):
    bn = ax_ref.shape[0]
    i = pl.program_id(0)
    rows = lax.broadcasted_iota(jnp.int32, (bn, 128), 0) + i * bn
    oneh = (rows == bounds_ref[...]).astype(jnp.float32)
    num = num_ref[...] + jnp.dot(oneh, cn_ref[...],
                                 preferred_element_type=jnp.float32)
    den = den_ref[...] + jnp.dot(oneh, cd_ref[...],
                                 preferred_element_type=jnp.float32)
    hn = ax_ref[...] + num / (den + _EPS_DEN)
    hn = g_ref[...] * hn * _BN_SCALE + b_ref[...]
    o_ref[...] = xin_ref[...] + jnp.maximum(hn, 0.0)


def _node_update(ax, num, den, bounds, x_in, g, b, bn):
    n, m = ax.shape
    cblk = CARRY0 // 128
    return pl.pallas_call(
        _node_update_kernel,
        grid=(n // bn,),
        in_specs=[
            pl.BlockSpec((bn, m), lambda i: (i, 0)),
            pl.BlockSpec((bn, m), lambda i: (i, 0)),
            pl.BlockSpec((bn, m), lambda i: (i, 0)),
            pl.BlockSpec((128, m), lambda i: (cblk, 0)),
            pl.BlockSpec((128, m), lambda i: (cblk, 0)),
            pl.BlockSpec((1, 128), lambda i: (0, 0)),
            pl.BlockSpec((bn, m), lambda i: (i, 0)),
            pl.BlockSpec((1, m), lambda i: (0, 0)),
            pl.BlockSpec((1, m), lambda i: (0, 0)),
        ],
        out_specs=pl.BlockSpec((bn, m), lambda i: (i, 0)),
        out_shape=jax.ShapeDtypeStruct((n, m), jnp.float32),
    )(ax, num, den, num, den, bounds, x_in, g.reshape(1, m), b.reshape(1, m))


def _edge_update_kernel(eij_ref, ein_ref, g_ref, b_ref, o_ref):
    en = g_ref[...] * eij_ref[...] * _BN_SCALE + b_ref[...]
    o_ref[...] = ein_ref[...] + jnp.maximum(en, 0.0)


def _edge_update(eij, e_in, g, b, bn):
    n, m = eij.shape
    return pl.pallas_call(
        _edge_update_kernel,
        grid=(n // bn,),
        in_specs=[
            pl.BlockSpec((bn, m), lambda i: (i, 0)),
            pl.BlockSpec((bn, m), lambda i: (i, 0)),
            pl.BlockSpec((1, m), lambda i: (0, 0)),
            pl.BlockSpec((1, m), lambda i: (0, 0)),
        ],
        out_specs=pl.BlockSpec((bn, m), lambda i: (i, 0)),
        out_shape=jax.ShapeDtypeStruct((n, m), jnp.float32),
    )(eij, e_in, g.reshape(1, m), b.reshape(1, m))


def _proj_kernel(x_ref, w_ref, b_ref, a_ref, bx_ref, d_ref, e_ref):
    res = (jnp.dot(x_ref[...], w_ref[...], preferred_element_type=jnp.float32)
           + b_ref[...])
    a_ref[...] = res[:, 0:128]
    bx_ref[...] = res[:, 128:256]
    d_ref[...] = res[:, 256:384]
    e_ref[...] = res[:, 384:512]


def _proj(x, w, b, bn):
    n, k = x.shape
    o = jax.ShapeDtypeStruct((n, 128), jnp.float32)
    return pl.pallas_call(
        _proj_kernel,
        grid=(n // bn,),
        in_specs=[
            pl.BlockSpec((bn, k), lambda i: (i, 0)),
            pl.BlockSpec((k, 512), lambda i: (0, 0)),
            pl.BlockSpec((1, 512), lambda i: (0, 0)),
        ],
        out_specs=[pl.BlockSpec((bn, 128), lambda i: (i, 0))] * 4,
        out_shape=[o, o, o, o],
    )(x, w, b.reshape(1, 512))


def _node_proj_kernel(ax_ref, num_ref, den_ref, cn_ref, cd_ref, bounds_ref,
                      xin_ref, g_ref, b_ref, w_ref, wb_ref,
                      h_ref, a_ref, bx_ref, d_ref, e_ref):
    bn = ax_ref.shape[0]
    i = pl.program_id(0)
    rows = lax.broadcasted_iota(jnp.int32, (bn, 128), 0) + i * bn
    oneh = (rows == bounds_ref[...]).astype(jnp.float32)
    num = num_ref[...] + jnp.dot(oneh, cn_ref[...],
                                 preferred_element_type=jnp.float32)
    den = den_ref[...] + jnp.dot(oneh, cd_ref[...],
                                 preferred_element_type=jnp.float32)
    hn = ax_ref[...] + num / (den + _EPS_DEN)
    hn = g_ref[...] * hn * _BN_SCALE + b_ref[...]
    hb = xin_ref[...] + jnp.maximum(hn, 0.0)
    h_ref[...] = hb
    res = (jnp.dot(hb, w_ref[...], preferred_element_type=jnp.float32)
           + wb_ref[...])
    a_ref[...] = res[:, 0:128]
    bx_ref[...] = res[:, 128:256]
    d_ref[...] = res[:, 256:384]
    e_ref[...] = res[:, 384:512]


def _node_proj(ax, num, den, bounds, x_in, g, b, w, wb, bn):
    n, m = ax.shape
    cblk = CARRY0 // 128
    o = jax.ShapeDtypeStruct((n, 128), jnp.float32)
    return pl.pallas_call(
        _node_proj_kernel,
        grid=(n // bn,),
        in_specs=[
            pl.BlockSpec((bn, m), lambda i: (i, 0)),
            pl.BlockSpec((bn, m), lambda i: (i, 0)),
            pl.BlockSpec((bn, m), lambda i: (i, 0)),
            pl.BlockSpec((128, m), lambda i: (cblk, 0)),
            pl.BlockSpec((128, m), lambda i: (cblk, 0)),
            pl.BlockSpec((1, 128), lambda i: (0, 0)),
            pl.BlockSpec((bn, m), lambda i: (i, 0)),
            pl.BlockSpec((1, m), lambda i: (0, 0)),
            pl.BlockSpec((1, m), lambda i: (0, 0)),
            pl.BlockSpec((m, 512), lambda i: (0, 0)),
            pl.BlockSpec((1, 512), lambda i: (0, 0)),
        ],
        out_specs=[pl.BlockSpec((bn, 128), lambda i: (i, 0))] * 5,
        out_shape=[o, o, o, o, o],
    )(ax, num, den, num, den, bounds, x_in, g.reshape(1, m), b.reshape(1, m),
      w, wb.reshape(1, 512))


def _ce_fused_kernel(eij_ref, ein_ref, g_ref, b_ref, w_ref, wb_ref,
                     e_ref, ce_ref):
    en = g_ref[...] * eij_ref[...] * _BN_SCALE + b_ref[...]
    enew = ein_ref[...] + jnp.maximum(en, 0.0)
    e_ref[...] = enew
    ce_ref[...] = (jnp.dot(enew, w_ref[...],
                           preferred_element_type=jnp.float32) + wb_ref[...])


def _ce_fused(eij, e_in, g, b, w, wb, bn):
    n, m = eij.shape
    o = jax.ShapeDtypeStruct((n, m), jnp.float32)
    return pl.pallas_call(
        _ce_fused_kernel,
        grid=(n // bn,),
        in_specs=[
            pl.BlockSpec((bn, m), lambda i: (i, 0)),
            pl.BlockSpec((bn, m), lambda i: (i, 0)),
            pl.BlockSpec((1, m), lambda i: (0, 0)),
            pl.BlockSpec((1, m), lambda i: (0, 0)),
            pl.BlockSpec((m, m), lambda i: (0, 0)),
            pl.BlockSpec((1, m), lambda i: (0, 0)),
        ],
        out_specs=[pl.BlockSpec((bn, m), lambda i: (i, 0))] * 2,
        out_shape=[o, o],
    )(eij, e_in, g.reshape(1, m), b.reshape(1, m), w, wb.reshape(1, m))


NPAIR = NCH // 2


def _sc_edge_body(dstp, srcp, permp, segidxp, keepp, opp, dx, ex, bx, ce,
                  eij_hbm, num_hbm, den_hbm, *scr):
    dst_v = scr[0:2]
    src_v = scr[2:4]
    pg = scr[4:6]
    pv = scr[6:8]
    xv = scr[8:10]
    kb = scr[10:12]
    ob = scr[12:14]
    d_b = scr[14:16]
    e_b = scr[16:18]
    b_b = scr[18:20]
    ce_b = scr[20:22]
    eij_b = scr[22:24]
    sn_b = scr[24:26]
    sd_b = scr[26:28]
    zero_buf = scr[28]
    zidx_v = scr[29]
    bbuf = scr[30]
    gsem = scr[31:33]
    ssem = scr[33:35]
    psem = scr[35:37]
    qsem = scr[37:39]

    t = lax.axis_index("s") * 2 + lax.axis_index("c")
    lanes = lax.iota(jnp.int32, 16)

    pltpu.sync_copy(dstp.at[pl.ds(t * EPT, 16)], bbuf)
    b_t = bbuf[pl.ds(0, 16)][0]
    pltpu.sync_copy(dstp.at[pl.ds((t + 1) * EPT, 16)], bbuf)
    b_next = bbuf[pl.ds(0, 16)][0]

    zvec = jnp.zeros((16,), jnp.float32)
    for r in range(ZB):
        for j in range(HV):
            zero_buf[r, pl.ds(j * 16, 16)] = zvec

    dump = DUMP0 + t
    zlo = jnp.where(t == 0, 0, b_t)
    zhi = b_next
    ntrips = (zhi - zlo + (ZB - 1)) // ZB

    def _ztrip(i, _):
        base_row = zlo + i * ZB
        for r in range(ZB // 16):
            rows = base_row + r * 16 + lanes
            rows = jnp.where(rows < zhi, rows, dump)
            zidx_v[pl.ds(r * 16, 16)] = rows
        pltpu.sync_copy(zero_buf, num_hbm.at[zidx_v])
        pltpu.sync_copy(zero_buf, den_hbm.at[zidx_v])
        return 0

    lax.fori_loop(0, ntrips, _ztrip, 0)

    crows = (CARRY0 + t, CARRY0 + 32 + t, CARRY0 + 64 + t, CARRY0 + 96 + t)
    for r in range(ZB // 16):
        rows = jnp.full((16,), dump, jnp.int32)
        if r == 0:
            for li, cr in enumerate(crows):
                rows = jnp.where(lanes == li, cr, rows)
        zidx_v[pl.ds(r * 16, 16)] = rows
    pltpu.sync_copy(zero_buf, num_hbm.at[zidx_v])
    pltpu.sync_copy(zero_buf, den_hbm.at[zidx_v])

    e0 = t * EPT

    def _issue_gidx(c, P, sem):
        base = e0 + c * K
        pltpu.async_copy(dstp.at[pl.ds(base, K)], dst_v[P], sem)
        pltpu.async_copy(srcp.at[pl.ds(base, K)], src_v[P], sem)
        pltpu.async_copy(keepp.at[pl.ds(base, K)], kb[P].at[pl.ds(0, K)], sem)
        pltpu.async_copy(opp.at[pl.ds(base, K)], ob[P].at[pl.ds(0, K)], sem)

    def _wait_gidx(c, P, sem):
        base = e0 + c * K
        pltpu.make_async_copy(dstp.at[pl.ds(base, K)], dst_v[P], sem).wait()
        pltpu.make_async_copy(srcp.at[pl.ds(base, K)], src_v[P], sem).wait()
        pltpu.make_async_copy(keepp.at[pl.ds(base, K)],
                              kb[P].at[pl.ds(0, K)], sem).wait()
        pltpu.make_async_copy(opp.at[pl.ds(base, K)],
                              ob[P].at[pl.ds(0, K)], sem).wait()

    def _issue_gathers(c, P):
        base = e0 + c * K
        pltpu.async_copy(dx.at[dst_v[P]], d_b[P], gsem[P])
        pltpu.async_copy(ex.at[src_v[P]], e_b[P], gsem[P])
        pltpu.async_copy(bx.at[src_v[P]], b_b[P], gsem[P])
        pltpu.async_copy(ce.at[pl.ds(base, K)], ce_b[P], gsem[P])

    def _wait_gathers(c, P):
        base = e0 + c * K
        pltpu.make_async_copy(dx.at[dst_v[P]], d_b[P], gsem[P]).wait()
        pltpu.make_async_copy(ex.at[src_v[P]], e_b[P], gsem[P]).wait()
        pltpu.make_async_copy(bx.at[src_v[P]], b_b[P], gsem[P]).wait()
        pltpu.make_async_copy(ce.at[pl.ds(base, K)], ce_b[P], gsem[P]).wait()

    def _wait_scatters(c, P):
        base = e0 + c * K
        pltpu.make_async_copy(eij_b[P], eij_hbm.at[pl.ds(base, K)],
                              ssem[P]).wait()
        pltpu.make_async_copy(sn_b[P].at[pl.ds(0, K)], num_hbm.at[xv[P]],
                              ssem[P]).wait()
        pltpu.make_async_copy(sd_b[P].at[pl.ds(0, K)], den_hbm.at[xv[P]],
                              ssem[P]).wait()

    def _compute(carry, P):
        def _edge(e, ec):
            an = ec[:HV]
            ad = ec[HV:]
            keep = lax.convert_element_type(kb[P][pl.ds(e, 16)][0],
                                            jnp.float32)
            o = ob[P][pl.ds(e, 16)][0]
            sls = [pl.ds(j * 16, 16) for j in range(HV)]
            dv = [d_b[P][e, sl] for sl in sls]
            ev = [e_b[P][e, sl] for sl in sls]
            cv = [ce_b[P][e, sl] for sl in sls]
            bv = [b_b[P][e, sl] for sl in sls]
            eij = [dv[j] + ev[j] + cv[j] for j in range(HV)]
            for j in range(HV):
                eij_b[P][e, sls[j]] = eij[j]
            sg = [1.0 / (1.0 + jnp.exp(-eij[j])) for j in range(HV)]
            na = [an[j] * keep + sg[j] * bv[j] for j in range(HV)]
            nd = [ad[j] * keep + sg[j] for j in range(HV)]
            for j in range(HV):
                sn_b[P][o, sls[j]] = na[j]
            for j in range(HV):
                sd_b[P][o, sls[j]] = nd[j]
            return tuple(na) + tuple(nd)

        return plsc.parallel_loop(0, K, unroll=2, carry=carry)(_edge)

    def _step(c, i, P, carry, pair, last_pack):
        Q = 1 - P
        base = e0 + c * K
        _wait_gathers(c, P)
        if pair:
            @pl.when(i > 0)
            def _():
                _wait_scatters(c - 2, P)
        else:
            _wait_scatters(c - 2, P)
        pltpu.async_copy(segidxp.at[pl.ds(base, K)], xv[P], qsem[P])
        if pair:
            _wait_gidx(c + 1, Q, psem[Q])
            _issue_gathers(c + 1, Q)
        carry = _compute(carry, P)
        if pair:
            if last_pack:
                _issue_gidx(c + 2, P, psem[P])
            else:
                @pl.when(i < NPAIR - 1)
                def _():
                    _issue_gidx(c + 2, P, psem[P])
        pltpu.make_async_copy(segidxp.at[pl.ds(base, K)], xv[P],
                              qsem[P]).wait()
        pltpu.async_copy(eij_b[P], eij_hbm.at[pl.ds(base, K)], ssem[P])
        pltpu.async_copy(sn_b[P].at[pl.ds(0, K)], num_hbm.at[xv[P]], ssem[P])
        pltpu.async_copy(sd_b[P].at[pl.ds(0, K)], den_hbm.at[xv[P]], ssem[P])
        return carry

    _issue_gidx(0, 0, psem[0])
    _wait_gidx(0, 0, psem[0])
    _issue_gathers(0, 0)
    _issue_gidx(1, 1, psem[1])

    zv = jnp.zeros((16,), jnp.float32)
    carry0 = (zv,) * (2 * HV)

    def _pair(i, carry):
        c0 = 2 * i
        carry = _step(c0, i, 0, carry, pair=True, last_pack=True)
        carry = _step(c0 + 1, i, 1, carry, pair=True, last_pack=False)
        return carry

    carry = lax.fori_loop(0, NPAIR, _pair, carry0)
    _step(NCH - 1, NPAIR, 0, carry, pair=False, last_pack=False)
    _wait_scatters(NCH - 2, 1)
    _wait_scatters(NCH - 1, 0)


_SC_SCRATCH = (
    [pltpu.VMEM((K,), jnp.int32) for _ in range(10)]
    + [pltpu.VMEM((64,), jnp.int32) for _ in range(4)]
    + [pltpu.VMEM((K, H), jnp.float32) for _ in range(10)]
    + [pltpu.VMEM((K + 1, H), jnp.float32) for _ in range(4)]
    + [pltpu.VMEM((ZB, H), jnp.float32)]
    + [pltpu.VMEM((ZB,), jnp.int32)]
    + [pltpu.VMEM((16,), jnp.int32)]
    + [pltpu.SemaphoreType.DMA for _ in range(8)]
)


@functools.partial(
    pl.kernel,
    mesh=plsc.VectorSubcoreMesh(core_axis_name="c", subcore_axis_name="s"),
    out_type=[
        jax.ShapeDtypeStruct((E, H), jnp.float32),
        jax.ShapeDtypeStruct((NROWS2, H), jnp.float32),
        jax.ShapeDtypeStruct((NROWS2, H), jnp.float32),
    ],
    scratch_types=_SC_SCRATCH,
)
def _sc_edge(*args):
    _sc_edge_body(*args)


def kernel(x, edge_attr, edge_index, node_W, node_b, edge_W, edge_b,
           A_W, A_b, B_W, B_b, C_W, C_b, D_W, D_b, E_W, E_b,
           bnx_g, bnx_b, bne_g, bne_b):
    src = edge_index[0]
    dst = edge_index[1]
    n = x.shape[0]
    L = A_W.shape[0]

    perm = jnp.argsort(dst)
    dst_s = dst[perm]
    src_s = src[perm]
    padN = jnp.full((16,), n, dtype=jnp.int32)
    dstp = jnp.concatenate([dst_s, padN])
    srcp = jnp.concatenate([src_s, padN])
    permp = jnp.concatenate([perm.astype(jnp.int32), padN])
    b33 = jnp.concatenate([dst_s[::EPT], jnp.full((1,), n, jnp.int32)])
    maskN = jnp.full((32,), n, jnp.int32)
    bounds = jnp.concatenate([b33[:32], maskN, b33[1:33], maskN]).reshape(1, 128)

    ii = jnp.arange(E, dtype=jnp.int32)
    prev_d = jnp.concatenate([jnp.full((1,), -1, jnp.int32), dst_s[:-1]])
    keep = (dst_s == prev_d).astype(jnp.int32)
    m = ((dst_s != prev_d) & (ii % K != 0)).astype(jnp.int32)
    o = jnp.cumsum(m.reshape(E // K, K), axis=1).reshape(E).astype(jnp.int32)
    tvec = ii // EPT
    bt = b33[tvec]
    bnx = b33[tvec + 1]
    tgt = jnp.where(dst_s == bt, CARRY0 + tvec,
                    jnp.where(dst_s == bnx, CARRY0 + 64 + tvec, dst_s))
    cend = (ii // K + 1) * K
    cont = (cend % EPT != 0) & (dst_s[jnp.minimum(cend, E - 1)] == dst_s)
    tgt = jnp.where(cont, DUMP0 + tvec, tgt)
    segidx = jnp.full((E,), 0, jnp.int32).at[ii - ii % K + o].set(tgt)
    dumped = jnp.full((E,), -1, jnp.int32).at[ii - ii % K + o].set(ii)
    segidx = jnp.where(dumped >= 0, segidx, NROWS + tvec * K + ii % K)
    nxt_d = jnp.concatenate([dst_s[1:], jnp.full((1,), -1, jnp.int32)])
    is_last = (dst_s != nxt_d) | (ii % K == K - 1)
    ostore = jnp.where(is_last, o, K)

    h = _mm(x, node_W, node_b, 2000)
    e = _mm(edge_attr[perm], edge_W, edge_b, 2000)
    invp = jnp.zeros((E,), jnp.int32).at[perm].set(ii)

    W_cat = [jnp.concatenate([A_W[l], B_W[l], D_W[l], E_W[l]], axis=1)
             for l in range(L)]
    b_cat = [jnp.concatenate([A_b[l], B_b[l], D_b[l], E_b[l]], axis=0)
             for l in range(L)]

    Ax = Bx = Dx = Ex = eij = num = den = None
    for l in range(L):
        if l == 0:
            Ax, Bx, Dx, Ex = _proj(h, W_cat[0], b_cat[0], 2000)
            Ce = _mm(e, C_W[0], C_b[0], 2000)
        else:
            h, Ax, Bx, Dx, Ex = _node_proj(
                Ax, num, den, bounds, h, bnx_g[l - 1], bnx_b[l - 1],
                W_cat[l], b_cat[l], 2000)
            e, Ce = _ce_fused(eij, e, bne_g[l - 1], bne_b[l - 1],
                              C_W[l], C_b[l], 2000)

        eij, num, den = _sc_edge(dstp, srcp, permp, segidx, keep, ostore,
                                 Dx, Ex, Bx, Ce)

    h = _node_update(Ax, num, den, bounds, h, bnx_g[L - 1], bnx_b[L - 1], 2000)
    e = _edge_update(eij, e, bne_g[L - 1], bne_b[L - 1], 2000)
    return (h, e[invp])


# speedup vs baseline: 1.0497x; 1.0497x over previous
import functools

import jax
import jax.numpy as jnp
from jax import lax
from jax.experimental import pallas as pl
from jax.experimental.pallas import tpu as pltpu
from jax.experimental.pallas import tpu_sc as plsc

_EPS_BN = 1e-5
_EPS_DEN = 1e-6
_BN_SCALE = (1.0 / (1.0 + _EPS_BN)) ** 0.5

N = 10000
E = 160000
H = 128
NT = 32
EPT = E // NT
K = 40
NCH = EPT // K
DUMP0 = N
CARRY0 = ((N + 32 + 127) // 128) * 128
NROWS = CARRY0 + 128
NROWS2 = NROWS + NT * K
HV = H // 16
ZB = 64


def _mm_kernel(x_ref, w_ref, b_ref, o_ref):
    o_ref[...] = (
        jnp.dot(x_ref[...], w_ref[...], preferred_element_type=jnp.float32)
        + b_ref[...]
    )


def _mm(x, w, b, bn):
    n, k = x.shape
    m = w.shape[1]
    return pl.pallas_call(
        _mm_kernel,
        grid=(n // bn,),
        in_specs=[
            pl.BlockSpec((bn, k), lambda i: (i, 0)),
            pl.BlockSpec((k, m), lambda i: (0, 0)),
            pl.BlockSpec((1, m), lambda i: (0, 0)),
        ],
        out_specs=pl.BlockSpec((bn, m), lambda i: (i, 0)),
        out_shape=jax.ShapeDtypeStruct((n, m), jnp.float32),
    )(x, w, b.reshape(1, m))


def _node_update_kernel(ax_ref, num_ref, den_ref, cn_ref, cd_ref, bounds_ref,
                        xin_ref, g_ref, b_ref, o_ref):
    bn = ax_ref.shape[0]
    i = pl.program_id(0)
    rows = lax.broadcasted_iota(jnp.int32, (bn, 128), 0) + i * bn
    oneh = (rows == bounds_ref[...]).astype(jnp.float32)
    num = num_ref[...] + jnp.dot(oneh, cn_ref[...],
                                 preferred_element_type=jnp.float32)
    den = den_ref[...] + jnp.dot(oneh, cd_ref[...],
                                 preferred_element_type=jnp.float32)
    hn = ax_ref[...] + num / (den + _EPS_DEN)
    hn = g_ref[...] * hn * _BN_SCALE + b_ref[...]
    o_ref[...] = xin_ref[...] + jnp.maximum(hn, 0.0)


def _node_update(ax, num, den, bounds, x_in, g, b, bn):
    n, m = ax.shape
    cblk = CARRY0 // 128
    return pl.pallas_call(
        _node_update_kernel,
        grid=(n // bn,),
        in_specs=[
            pl.BlockSpec((bn, m), lambda i: (i, 0)),
            pl.BlockSpec((bn, m), lambda i: (i, 0)),
            pl.BlockSpec((bn, m), lambda i: (i, 0)),
            pl.BlockSpec((128, m), lambda i: (cblk, 0)),
            pl.BlockSpec((128, m), lambda i: (cblk, 0)),
            pl.BlockSpec((1, 128), lambda i: (0, 0)),
            pl.BlockSpec((bn, m), lambda i: (i, 0)),
            pl.BlockSpec((1, m), lambda i: (0, 0)),
            pl.BlockSpec((1, m), lambda i: (0, 0)),
        ],
        out_specs=pl.BlockSpec((bn, m), lambda i: (i, 0)),
        out_shape=jax.ShapeDtypeStruct((n, m), jnp.float32),
    )(ax, num, den, num, den, bounds, x_in, g.reshape(1, m), b.reshape(1, m))


def _edge_update_kernel(eij_ref, ein_ref, g_ref, b_ref, o_ref):
    en = g_ref[...] * eij_ref[...] * _BN_SCALE + b_ref[...]
    o_ref[...] = ein_ref[...] + jnp.maximum(en, 0.0)


def _edge_update(eij, e_in, g, b, bn):
    n, m = eij.shape
    return pl.pallas_call(
        _edge_update_kernel,
        grid=(n // bn,),
        in_specs=[
            pl.BlockSpec((bn, m), lambda i: (i, 0)),
            pl.BlockSpec((bn, m), lambda i: (i, 0)),
            pl.BlockSpec((1, m), lambda i: (0, 0)),
            pl.BlockSpec((1, m), lambda i: (0, 0)),
        ],
        out_specs=pl.BlockSpec((bn, m), lambda i: (i, 0)),
        out_shape=jax.ShapeDtypeStruct((n, m), jnp.float32),
    )(eij, e_in, g.reshape(1, m), b.reshape(1, m))


def _proj_kernel(x_ref, w_ref, b_ref, a_ref, bx_ref, d_ref, e_ref):
    res = (jnp.dot(x_ref[...], w_ref[...], preferred_element_type=jnp.float32)
           + b_ref[...])
    a_ref[...] = res[:, 0:128]
    bx_ref[...] = res[:, 128:256]
    d_ref[...] = res[:, 256:384]
    e_ref[...] = res[:, 384:512]


def _proj(x, w, b, bn):
    n, k = x.shape
    o = jax.ShapeDtypeStruct((n, 128), jnp.float32)
    return pl.pallas_call(
        _proj_kernel,
        grid=(n // bn,),
        in_specs=[
            pl.BlockSpec((bn, k), lambda i: (i, 0)),
            pl.BlockSpec((k, 512), lambda i: (0, 0)),
            pl.BlockSpec((1, 512), lambda i: (0, 0)),
        ],
        out_specs=[pl.BlockSpec((bn, 128), lambda i: (i, 0))] * 4,
        out_shape=[o, o, o, o],
    )(x, w, b.reshape(1, 512))


def _node_proj_kernel(ax_ref, num_ref, den_ref, cn_ref, cd_ref, bounds_ref,
                      xin_ref, g_ref, b_ref, w_ref, wb_ref,
                      h_ref, a_ref, bx_ref, d_ref, e_ref):
    bn = ax_ref.shape[0]
    i = pl.program_id(0)
    rows = lax.broadcasted_iota(jnp.int32, (bn, 128), 0) + i * bn
    oneh = (rows == bounds_ref[...]).astype(jnp.float32)
    num = num_ref[...] + jnp.dot(oneh, cn_ref[...],
                                 preferred_element_type=jnp.float32)
    den = den_ref[...] + jnp.dot(oneh, cd_ref[...],
                                 preferred_element_type=jnp.float32)
    hn = ax_ref[...] + num / (den + _EPS_DEN)
    hn = g_ref[...] * hn * _BN_SCALE + b_ref[...]
    hb = xin_ref[...] + jnp.maximum(hn, 0.0)
    h_ref[...] = hb
    res = (jnp.dot(hb, w_ref[...], preferred_element_type=jnp.float32)
           + wb_ref[...])
    a_ref[...] = res[:, 0:128]
    bx_ref[...] = res[:, 128:256]
    d_ref[...] = res[:, 256:384]
    e_ref[...] = res[:, 384:512]


def _node_proj(ax, num, den, bounds, x_in, g, b, w, wb, bn):
    n, m = ax.shape
    cblk = CARRY0 // 128
    o = jax.ShapeDtypeStruct((n, 128), jnp.float32)
    return pl.pallas_call(
        _node_proj_kernel,
        grid=(n // bn,),
        in_specs=[
            pl.BlockSpec((bn, m), lambda i: (i, 0)),
            pl.BlockSpec((bn, m), lambda i: (i, 0)),
            pl.BlockSpec((bn, m), lambda i: (i, 0)),
            pl.BlockSpec((128, m), lambda i: (cblk, 0)),
            pl.BlockSpec((128, m), lambda i: (cblk, 0)),
            pl.BlockSpec((1, 128), lambda i: (0, 0)),
            pl.BlockSpec((bn, m), lambda i: (i, 0)),
            pl.BlockSpec((1, m), lambda i: (0, 0)),
            pl.BlockSpec((1, m), lambda i: (0, 0)),
            pl.BlockSpec((m, 512), lambda i: (0, 0)),
            pl.BlockSpec((1, 512), lambda i: (0, 0)),
        ],
        out_specs=[pl.BlockSpec((bn, 128), lambda i: (i, 0))] * 5,
        out_shape=[o, o, o, o, o],
    )(ax, num, den, num, den, bounds, x_in, g.reshape(1, m), b.reshape(1, m),
      w, wb.reshape(1, 512))


def _ce_fused_kernel(eij_ref, ein_ref, g_ref, b_ref, w_ref, wb_ref,
                     e_ref, ce_ref):
    en = g_ref[...] * eij_ref[...] * _BN_SCALE + b_ref[...]
    enew = ein_ref[...] + jnp.maximum(en, 0.0)
    e_ref[...] = enew
    ce_ref[...] = (jnp.dot(enew, w_ref[...],
                           preferred_element_type=jnp.float32) + wb_ref[...])


def _ce_fused(eij, e_in, g, b, w, wb, bn):
    n, m = eij.shape
    o = jax.ShapeDtypeStruct((n, m), jnp.float32)
    return pl.pallas_call(
        _ce_fused_kernel,
        grid=(n // bn,),
        in_specs=[
            pl.BlockSpec((bn, m), lambda i: (i, 0)),
            pl.BlockSpec((bn, m), lambda i: (i, 0)),
            pl.BlockSpec((1, m), lambda i: (0, 0)),
            pl.BlockSpec((1, m), lambda i: (0, 0)),
            pl.BlockSpec((m, m), lambda i: (0, 0)),
            pl.BlockSpec((1, m), lambda i: (0, 0)),
        ],
        out_specs=[pl.BlockSpec((bn, m), lambda i: (i, 0))] * 2,
        out_shape=[o, o],
    )(eij, e_in, g.reshape(1, m), b.reshape(1, m), w, wb.reshape(1, m))


NPAIR = NCH // 2


def _sc_edge_body(dstp, srcp, permp, segidxp, keepp, opp, dx, ex, bx, ce,
                  eij_hbm, num_hbm, den_hbm, *scr):
    dst_v = scr[0:2]
    src_v = scr[2:4]
    pg = scr[4:6]
    pv = scr[6:8]
    xv = scr[8:10]
    kb = scr[10:12]
    ob = scr[12:14]
    d_b = scr[14:16]
    e_b = scr[16:18]
    b_b = scr[18:20]
    ce_b = scr[20:22]
    eij_b = scr[22:24]
    sn_b = scr[24:26]
    sd_b = scr[26:28]
    zero_buf = scr[28]
    zidx_v = scr[29]
    bbuf = scr[30]
    gsem = scr[31:33]
    ssem = scr[33:35]
    psem = scr[35:37]
    qsem = scr[37:39]

    t = lax.axis_index("s") * 2 + lax.axis_index("c")
    lanes = lax.iota(jnp.int32, 16)

    pltpu.sync_copy(dstp.at[pl.ds(t * EPT, 16)], bbuf)
    b_t = bbuf[pl.ds(0, 16)][0]
    pltpu.sync_copy(dstp.at[pl.ds((t + 1) * EPT, 16)], bbuf)
    b_next = bbuf[pl.ds(0, 16)][0]

    zvec = jnp.zeros((16,), jnp.float32)
    for r in range(ZB):
        for j in range(HV):
            zero_buf[r, pl.ds(j * 16, 16)] = zvec

    dump = DUMP0 + t
    zlo = jnp.where(t == 0, 0, b_t)
    zhi = b_next
    ntrips = (zhi - zlo + (ZB - 1)) // ZB

    def _ztrip(i, _):
        base_row = zlo + i * ZB
        for r in range(ZB // 16):
            rows = base_row + r * 16 + lanes
            rows = jnp.where(rows < zhi, rows, dump)
            zidx_v[pl.ds(r * 16, 16)] = rows
        pltpu.sync_copy(zero_buf, num_hbm.at[zidx_v])
        pltpu.sync_copy(zero_buf, den_hbm.at[zidx_v])
        return 0

    lax.fori_loop(0, ntrips, _ztrip, 0)

    crows = (CARRY0 + t, CARRY0 + 32 + t, CARRY0 + 64 + t, CARRY0 + 96 + t)
    for r in range(ZB // 16):
        rows = jnp.full((16,), dump, jnp.int32)
        if r == 0:
            for li, cr in enumerate(crows):
                rows = jnp.where(lanes == li, cr, rows)
        zidx_v[pl.ds(r * 16, 16)] = rows
    pltpu.sync_copy(zero_buf, num_hbm.at[zidx_v])
    pltpu.sync_copy(zero_buf, den_hbm.at[zidx_v])

    e0 = t * EPT

    def _issue_gidx(c, P, sem):
        base = e0 + c * K
        pltpu.async_copy(dstp.at[pl.ds(base, K)], dst_v[P], sem)
        pltpu.async_copy(srcp.at[pl.ds(base, K)], src_v[P], sem)
        pltpu.async_copy(permp.at[pl.ds(base, K)], pg[P], sem)
        pltpu.async_copy(keepp.at[pl.ds(base, K)], kb[P].at[pl.ds(0, K)], sem)
        pltpu.async_copy(opp.at[pl.ds(base, K)], ob[P].at[pl.ds(0, K)], sem)

    def _wait_gidx(c, P, sem):
        base = e0 + c * K
        pltpu.make_async_copy(dstp.at[pl.ds(base, K)], dst_v[P], sem).wait()
        pltpu.make_async_copy(srcp.at[pl.ds(base, K)], src_v[P], sem).wait()
        pltpu.make_async_copy(permp.at[pl.ds(base, K)], pg[P], sem).wait()
        pltpu.make_async_copy(keepp.at[pl.ds(base, K)],
                              kb[P].at[pl.ds(0, K)], sem).wait()
        pltpu.make_async_copy(opp.at[pl.ds(base, K)],
                              ob[P].at[pl.ds(0, K)], sem).wait()

    def _issue_gathers(P):
        pltpu.async_copy(dx.at[dst_v[P]], d_b[P], gsem[P])
        pltpu.async_copy(ex.at[src_v[P]], e_b[P], gsem[P])
        pltpu.async_copy(bx.at[src_v[P]], b_b[P], gsem[P])
        pltpu.async_copy(ce.at[pg[P]], ce_b[P], gsem[P])

    def _wait_gathers(P):
        pltpu.make_async_copy(dx.at[dst_v[P]], d_b[P], gsem[P]).wait()
        pltpu.make_async_copy(ex.at[src_v[P]], e_b[P], gsem[P]).wait()
        pltpu.make_async_copy(bx.at[src_v[P]], b_b[P], gsem[P]).wait()
        pltpu.make_async_copy(ce.at[pg[P]], ce_b[P], gsem[P]).wait()

    def _wait_scatters(P):
        pltpu.make_async_copy(eij_b[P], eij_hbm.at[pv[P]], ssem[P]).wait()
        pltpu.make_async_copy(sn_b[P].at[pl.ds(0, K)], num_hbm.at[xv[P]],
                              ssem[P]).wait()
        pltpu.make_async_copy(sd_b[P].at[pl.ds(0, K)], den_hbm.at[xv[P]],
                              ssem[P]).wait()

    def _compute(carry, P):
        def _edge(e, ec):
            an = ec[:HV]
            ad = ec[HV:]
            keep = lax.convert_element_type(kb[P][pl.ds(e, 16)][0],
                                            jnp.float32)
            o = ob[P][pl.ds(e, 16)][0]
            sls = [pl.ds(j * 16, 16) for j in range(HV)]
            dv = [d_b[P][e, sl] for sl in sls]
            ev = [e_b[P][e, sl] for sl in sls]
            cv = [ce_b[P][e, sl] for sl in sls]
            bv = [b_b[P][e, sl] for sl in sls]
            eij = [dv[j] + ev[j] + cv[j] for j in range(HV)]
            for j in range(HV):
                eij_b[P][e, sls[j]] = eij[j]
            sg = [1.0 / (1.0 + jnp.exp(-eij[j])) for j in range(HV)]
            na = [an[j] * keep + sg[j] * bv[j] for j in range(HV)]
            nd = [ad[j] * keep + sg[j] for j in range(HV)]
            for j in range(HV):
                sn_b[P][o, sls[j]] = na[j]
            for j in range(HV):
                sd_b[P][o, sls[j]] = nd[j]
            return tuple(na) + tuple(nd)

        return plsc.parallel_loop(0, K, unroll=2, carry=carry)(_edge)

    def _step(c, i, P, carry, pair, last_pack):
        Q = 1 - P
        base = e0 + c * K
        _wait_gathers(P)
        if pair:
            @pl.when(i > 0)
            def _():
                _wait_scatters(P)
        else:
            _wait_scatters(P)
        pltpu.async_copy(permp.at[pl.ds(base, K)], pv[P], qsem[P])
        pltpu.async_copy(segidxp.at[pl.ds(base, K)], xv[P], qsem[P])
        if pair:
            _wait_gidx(c + 1, Q, psem[Q])
            _issue_gathers(Q)
        carry = _compute(carry, P)
        if pair:
            if last_pack:
                _issue_gidx(c + 2, P, psem[P])
            else:
                @pl.when(i < NPAIR - 1)
                def _():
                    _issue_gidx(c + 2, P, psem[P])
        pltpu.make_async_copy(permp.at[pl.ds(base, K)], pv[P], qsem[P]).wait()
        pltpu.make_async_copy(segidxp.at[pl.ds(base, K)], xv[P],
                              qsem[P]).wait()
        pltpu.async_copy(eij_b[P], eij_hbm.at[pv[P]], ssem[P])
        pltpu.async_copy(sn_b[P].at[pl.ds(0, K)], num_hbm.at[xv[P]], ssem[P])
        pltpu.async_copy(sd_b[P].at[pl.ds(0, K)], den_hbm.at[xv[P]], ssem[P])
        return carry

    _issue_gidx(0, 0, psem[0])
    _wait_gidx(0, 0, psem[0])
    _issue_gathers(0)
    _issue_gidx(1, 1, psem[1])

    zv = jnp.zeros((16,), jnp.float32)
    carry0 = (zv,) * (2 * HV)

    def _pair(i, carry):
        c0 = 2 * i
        carry = _step(c0, i, 0, carry, pair=True, last_pack=True)
        carry = _step(c0 + 1, i, 1, carry, pair=True, last_pack=False)
        return carry

    carry = lax.fori_loop(0, NPAIR, _pair, carry0)
    _step(NCH - 1, NPAIR, 0, carry, pair=False, last_pack=False)
    _wait_scatters(1)
    _wait_scatters(0)


_SC_SCRATCH = (
    [pltpu.VMEM((K,), jnp.int32) for _ in range(10)]
    + [pltpu.VMEM((64,), jnp.int32) for _ in range(4)]
    + [pltpu.VMEM((K, H), jnp.float32) for _ in range(10)]
    + [pltpu.VMEM((K + 1, H), jnp.float32) for _ in range(4)]
    + [pltpu.VMEM((ZB, H), jnp.float32)]
    + [pltpu.VMEM((ZB,), jnp.int32)]
    + [pltpu.VMEM((16,), jnp.int32)]
    + [pltpu.SemaphoreType.DMA for _ in range(8)]
)


@functools.partial(
    pl.kernel,
    mesh=plsc.VectorSubcoreMesh(core_axis_name="c", subcore_axis_name="s"),
    out_type=[
        jax.ShapeDtypeStruct((E, H), jnp.float32),
        jax.ShapeDtypeStruct((NROWS2, H), jnp.float32),
        jax.ShapeDtypeStruct((NROWS2, H), jnp.float32),
    ],
    scratch_types=_SC_SCRATCH,
)
def _sc_edge(*args):
    _sc_edge_body(*args)


def kernel(x, edge_attr, edge_index, node_W, node_b, edge_W, edge_b,
           A_W, A_b, B_W, B_b, C_W, C_b, D_W, D_b, E_W, E_b,
           bnx_g, bnx_b, bne_g, bne_b):
    src = edge_index[0]
    dst = edge_index[1]
    n = x.shape[0]
    L = A_W.shape[0]

    perm = jnp.argsort(dst)
    dst_s = dst[perm]
    src_s = src[perm]
    padN = jnp.full((16,), n, dtype=jnp.int32)
    dstp = jnp.concatenate([dst_s, padN])
    srcp = jnp.concatenate([src_s, padN])
    permp = jnp.concatenate([perm.astype(jnp.int32), padN])
    b33 = jnp.concatenate([dst_s[::EPT], jnp.full((1,), n, jnp.int32)])
    maskN = jnp.full((32,), n, jnp.int32)
    bounds = jnp.concatenate([b33[:32], maskN, b33[1:33], maskN]).reshape(1, 128)

    ii = jnp.arange(E, dtype=jnp.int32)
    prev_d = jnp.concatenate([jnp.full((1,), -1, jnp.int32), dst_s[:-1]])
    keep = (dst_s == prev_d).astype(jnp.int32)
    m = ((dst_s != prev_d) & (ii % K != 0)).astype(jnp.int32)
    o = jnp.cumsum(m.reshape(E // K, K), axis=1).reshape(E).astype(jnp.int32)
    tvec = ii // EPT
    bt = b33[tvec]
    bnx = b33[tvec + 1]
    tgt = jnp.where(dst_s == bt, CARRY0 + tvec,
                    jnp.where(dst_s == bnx, CARRY0 + 64 + tvec, dst_s))
    cend = (ii // K + 1) * K
    cont = (cend % EPT != 0) & (dst_s[jnp.minimum(cend, E - 1)] == dst_s)
    tgt = jnp.where(cont, DUMP0 + tvec, tgt)
    segidx = jnp.full((E,), 0, jnp.int32).at[ii - ii % K + o].set(tgt)
    dumped = jnp.full((E,), -1, jnp.int32).at[ii - ii % K + o].set(ii)
    segidx = jnp.where(dumped >= 0, segidx, NROWS + tvec * K + ii % K)
    nxt_d = jnp.concatenate([dst_s[1:], jnp.full((1,), -1, jnp.int32)])
    is_last = (dst_s != nxt_d) | (ii % K == K - 1)
    ostore = jnp.where(is_last, o, K)

    h = _mm(x, node_W, node_b, 2000)
    e = _mm(edge_attr, edge_W, edge_b, 2000)

    W_cat = [jnp.concatenate([A_W[l], B_W[l], D_W[l], E_W[l]], axis=1)
             for l in range(L)]
    b_cat = [jnp.concatenate([A_b[l], B_b[l], D_b[l], E_b[l]], axis=0)
             for l in range(L)]

    Ax = Bx = Dx = Ex = eij = num = den = None
    for l in range(L):
        if l == 0:
            Ax, Bx, Dx, Ex = _proj(h, W_cat[0], b_cat[0], 2000)
            Ce = _mm(e, C_W[0], C_b[0], 2000)
        else:
            h, Ax, Bx, Dx, Ex = _node_proj(
                Ax, num, den, bounds, h, bnx_g[l - 1], bnx_b[l - 1],
                W_cat[l], b_cat[l], 2000)
            e, Ce = _ce_fused(eij, e, bne_g[l - 1], bne_b[l - 1],
                              C_W[l], C_b[l], 2000)

        eij, num, den = _sc_edge(dstp, srcp, permp, segidx, keep, ostore,
                                 Dx, Ex, Bx, Ce)

    h = _node_update(Ax, num, den, bounds, h, bnx_g[L - 1], bnx_b[L - 1], 2000)
    e = _edge_update(eij, e, bne_g[L - 1], bne_b[L - 1], 2000)
    return (h, e)
